# Initial kernel scaffold; baseline (speedup 1.0000x reference)
import jax
import jax.numpy as jnp
from jax.experimental import pallas as pl


def kernel(H, B, x0, i, j):
    raise NotImplementedError("write your pallas kernel here")


# speedup vs baseline: 166.1542x; 166.1542x over previous
import functools

import jax
import jax.numpy as jnp
from jax import lax
from jax.experimental import pallas as pl
from jax.experimental.pallas import tpu as pltpu
from jax.experimental.pallas import tpu_sc as plsc

N_NODES = 50000
NP = 2 * N_NODES
NP2 = 100352
SLICE = NP2 // 16
NW = 32
S = 1024
GROUPS = S // 16
KROWS = S // 128
CG_ITERS_K = 32


def _make_sc_matvec(nsub):
    mesh = plsc.VectorSubcoreMesh(core_axis_name="c", subcore_axis_name="s")

    @functools.partial(
        pl.kernel,
        out_type=jax.ShapeDtypeStruct((2, NP2), jnp.float32),
        mesh=mesh,
        compiler_params=pltpu.CompilerParams(needs_layout_passes=False),
        scratch_types=[
            pltpu.VMEM((NP2,), jnp.float32),
            pltpu.VMEM((S,), jnp.int32),
            pltpu.VMEM((KROWS, 128), jnp.int32),
            pltpu.VMEM((KROWS, 128), jnp.int32),
            pltpu.VMEM((S,), jnp.float32),
            pltpu.VMEM((S,), jnp.float32),
            pltpu.VMEM((S,), jnp.float32),
            pltpu.VMEM((S,), jnp.float32),
            pltpu.VMEM((S,), jnp.float32),
            pltpu.VMEM((S,), jnp.float32),
            pltpu.VMEM((SLICE,), jnp.float32),
            pltpu.VMEM_SHARED((NP2,), jnp.float32),
            pltpu.SemaphoreType.DMA,
        ],
    )
    def sc_matvec(v_hbm, j2_hbm, i2_hbm, i2p1_hbm,
                  h00_hbm, h01_hbm, h10_hbm, h11_hbm, out_hbm,
                  v_v, j2_v, i2_v, i2p1_v, h00_v, h01_v, h10_v, h11_v,
                  val0_v, val1_v, buf_v, acc, sem):
        cid = lax.axis_index("c")
        sid = lax.axis_index("s")
        wid = cid * 16 + sid
        ep = nsub * S

        def zbody(k, c):
            buf_v[pl.ds(k * 16, 16)] = jnp.zeros((16,), jnp.float32)
            return c
        lax.fori_loop(0, SLICE // 16, zbody, 0)
        pltpu.sync_copy(buf_v, acc.at[pl.ds(sid * SLICE, SLICE)])
        pltpu.sync_copy(v_hbm, v_v)
        plsc.subcore_barrier()

        def sub_body(sub, c):
            base = pl.multiple_of(wid * ep + sub * S, S)
            row = pl.multiple_of((wid * ep + sub * S) // 128, KROWS)
            cps = [
                pltpu.async_copy(j2_hbm.at[pl.ds(base, S)], j2_v, sem),
                pltpu.async_copy(i2_hbm.at[pl.ds(row, KROWS)], i2_v, sem),
                pltpu.async_copy(i2p1_hbm.at[pl.ds(row, KROWS)], i2p1_v, sem),
                pltpu.async_copy(h00_hbm.at[pl.ds(base, S)], h00_v, sem),
                pltpu.async_copy(h01_hbm.at[pl.ds(base, S)], h01_v, sem),
                pltpu.async_copy(h10_hbm.at[pl.ds(base, S)], h10_v, sem),
                pltpu.async_copy(h11_hbm.at[pl.ds(base, S)], h11_v, sem),
            ]
            for cp in cps:
                cp.wait()

            def g_body(g, cc):
                o = g * 16
                jv = j2_v[pl.ds(o, 16)]
                vj0 = plsc.load_gather(v_v, [jv])
                vj1 = plsc.load_gather(v_v, [jv + 1])
                val0_v[pl.ds(o, 16)] = (h00_v[pl.ds(o, 16)] * vj0
                                        + h01_v[pl.ds(o, 16)] * vj1)
                val1_v[pl.ds(o, 16)] = (h10_v[pl.ds(o, 16)] * vj0
                                        + h11_v[pl.ds(o, 16)] * vj1)
                return cc
            lax.fori_loop(0, GROUPS, g_body, 0)

            def s_body(k, cc):
                pltpu.sync_copy(val0_v.at[pl.ds(k * 128, 128)],
                                acc.at[i2_v.at[k]], add=True)
                pltpu.sync_copy(val1_v.at[pl.ds(k * 128, 128)],
                                acc.at[i2p1_v.at[k]], add=True)
                return cc
            lax.fori_loop(0, KROWS, s_body, 0)
            return c
        lax.fori_loop(0, nsub, sub_body, 0)

        plsc.subcore_barrier()
        pltpu.sync_copy(acc.at[pl.ds(sid * SLICE, SLICE)], buf_v)
        pltpu.sync_copy(buf_v, out_hbm.at[cid, pl.ds(sid * SLICE, SLICE)])

    return sc_matvec


def _tc_init_body(apk2_ref, b_ref, r_ref, p_ref):
    r = apk2_ref[0] + apk2_ref[1] - b_ref[...]
    r_ref[...] = r
    p_ref[...] = -r


def _tc_init(apk2, b):
    shp = jax.ShapeDtypeStruct((NP2 // 128, 128), jnp.float32)
    return pl.pallas_call(_tc_init_body, out_shape=(shp, shp))(apk2, b)


def _tc_update_body(apk2_ref, pk_ref, rk_ref, xk_ref, xo_ref, ro_ref, po_ref):
    apk = apk2_ref[0] + apk2_ref[1]
    rk = rk_ref[...]
    pk = pk_ref[...]
    rkrk = jnp.sum(rk * rk)
    alpha = rkrk / jnp.sum(pk * apk)
    xo_ref[...] = xk_ref[...] + alpha * pk
    rn = rk + alpha * apk
    ro_ref[...] = rn
    beta = jnp.sum(rn * rn) / rkrk
    po_ref[...] = -rn + beta * pk


def _tc_update(apk2, pk, rk, xk):
    shp = jax.ShapeDtypeStruct((NP2 // 128, 128), jnp.float32)
    return pl.pallas_call(_tc_update_body,
                          out_shape=(shp, shp, shp))(apk2, pk, rk, xk)


def kernel(H, B, x0, i, j):
    f32 = jnp.float32
    e = i.shape[0]
    per_w = -(-e // (NW * S)) * S
    e_pad = per_w * NW
    nsub = per_w // S
    pad = e_pad - e

    h00 = jnp.concatenate([H[:, 0, 0], jnp.zeros((pad,), f32)])
    h01 = jnp.concatenate([H[:, 0, 1], jnp.zeros((pad,), f32)])
    h10 = jnp.concatenate([H[:, 1, 0], jnp.zeros((pad,), f32)])
    h11 = jnp.concatenate([H[:, 1, 1], jnp.zeros((pad,), f32)])
    i2 = jnp.concatenate([2 * i, jnp.full((pad,), NP2 - 2, jnp.int32)])
    i2p1 = i2 + 1
    j2 = jnp.concatenate([2 * j, jnp.zeros((pad,), jnp.int32)])
    i2 = i2.reshape(e_pad // 128, 128)
    i2p1 = i2p1.reshape(e_pad // 128, 128)

    bp = jnp.pad(B, (0, NP2 - NP)).reshape(NP2 // 128, 128)
    x0p = jnp.pad(x0, (0, NP2 - NP))

    sc_matvec = _make_sc_matvec(nsub)

    def matvec(v):
        out = sc_matvec(v, j2, i2, i2p1, h00, h01, h10, h11)
        return out.reshape(2, NP2 // 128, 128)

    rk, pk = _tc_init(matvec(x0p), bp)
    xk = x0p.reshape(NP2 // 128, 128)

    def body(carry, _):
        xk, rk, pk = carry
        apk2 = matvec(pk.reshape(NP2))
        xk, rk, pk = _tc_update(apk2, pk, rk, xk)
        return (xk, rk, pk), None

    (xk, rk, pk), _ = lax.scan(body, (xk, rk, pk), None, length=CG_ITERS_K)
    return xk.reshape(NP2)[:NP]


# speedup vs baseline: 203.6429x; 1.2256x over previous
import functools

import jax
import jax.numpy as jnp
from jax import lax
from jax.experimental import pallas as pl
from jax.experimental.pallas import tpu as pltpu
from jax.experimental.pallas import tpu_sc as plsc

N_NODES = 50000
NP = 2 * N_NODES
NP2 = 100352
SLICE = NP2 // 16
NW = 32
S = 1024
GROUPS = S // 16
KROWS = S // 128
CG_ITERS_K = 32


def _make_sc_matvec(nsub):
    mesh = plsc.VectorSubcoreMesh(core_axis_name="c", subcore_axis_name="s")
    half = nsub // 2

    buf_set = [
        pltpu.VMEM((S,), jnp.int32),
        pltpu.VMEM((KROWS, 128), jnp.int32),
        pltpu.VMEM((KROWS, 128), jnp.int32),
        pltpu.VMEM((S,), jnp.float32),
        pltpu.VMEM((S,), jnp.float32),
        pltpu.VMEM((S,), jnp.float32),
        pltpu.VMEM((S,), jnp.float32),
        pltpu.VMEM((S,), jnp.float32),
        pltpu.VMEM((S,), jnp.float32),
    ]

    @functools.partial(
        pl.kernel,
        out_type=jax.ShapeDtypeStruct((2, NP2), jnp.float32),
        mesh=mesh,
        compiler_params=pltpu.CompilerParams(needs_layout_passes=False),
        scratch_types=[
            pltpu.VMEM((NP,), jnp.float32),
            *buf_set,
            *buf_set,
            pltpu.VMEM((SLICE,), jnp.float32),
            pltpu.VMEM_SHARED((NP2,), jnp.float32),
            pltpu.SemaphoreType.DMA,
            pltpu.SemaphoreType.DMA,
        ],
    )
    def sc_matvec(v_hbm, j2_hbm, i2_hbm, i2p1_hbm,
                  h00_hbm, h01_hbm, h10_hbm, h11_hbm, out_hbm,
                  v_v,
                  j2_a, i2_a, i2p1_a, h00_a, h01_a, h10_a, h11_a, v0_a, v1_a,
                  j2_b, i2_b, i2p1_b, h00_b, h01_b, h10_b, h11_b, v0_b, v1_b,
                  buf_v, acc, ssem, csem):
        cid = lax.axis_index("c")
        sid = lax.axis_index("s")
        wid = cid * 16 + sid
        ep = nsub * S

        bufs_a = (j2_a, i2_a, i2p1_a, h00_a, h01_a, h10_a, h11_a, v0_a, v1_a)
        bufs_b = (j2_b, i2_b, i2p1_b, h00_b, h01_b, h10_b, h11_b, v0_b, v1_b)

        def fire_stage(sub, bufs):
            j2_v, i2_v, i2p1_v, h00_v, h01_v, h10_v, h11_v, _, _ = bufs
            base = pl.multiple_of(wid * ep + sub * S, S)
            row = pl.multiple_of((wid * ep + sub * S) // 128, KROWS)
            pltpu.async_copy(j2_hbm.at[pl.ds(base, S)], j2_v, ssem)
            pltpu.async_copy(i2_hbm.at[pl.ds(row, KROWS)], i2_v, ssem)
            pltpu.async_copy(i2p1_hbm.at[pl.ds(row, KROWS)], i2p1_v, ssem)
            pltpu.async_copy(h00_hbm.at[pl.ds(base, S)], h00_v, ssem)
            pltpu.async_copy(h01_hbm.at[pl.ds(base, S)], h01_v, ssem)
            pltpu.async_copy(h10_hbm.at[pl.ds(base, S)], h10_v, ssem)
            pltpu.async_copy(h11_hbm.at[pl.ds(base, S)], h11_v, ssem)

        def drain_stage():
            cp = pltpu.make_async_copy(h00_hbm.at[pl.ds(0, S)], h00_a, ssem)
            for _ in range(7):
                cp.wait()

        def compute(bufs):
            j2_v, _, _, h00_v, h01_v, h10_v, h11_v, v0_v, v1_v = bufs

            def g_body(g, cc):
                o = g * 16
                jv = j2_v[pl.ds(o, 16)]
                vj0 = plsc.load_gather(v_v, [jv])
                vj1 = plsc.load_gather(v_v, [jv + 1])
                v0_v[pl.ds(o, 16)] = (h00_v[pl.ds(o, 16)] * vj0
                                      + h01_v[pl.ds(o, 16)] * vj1)
                v1_v[pl.ds(o, 16)] = (h10_v[pl.ds(o, 16)] * vj0
                                      + h11_v[pl.ds(o, 16)] * vj1)
                return cc
            lax.fori_loop(0, GROUPS, g_body, 0)

        def fire_scatter(bufs):
            _, i2_v, i2p1_v, _, _, _, _, v0_v, v1_v = bufs

            def sb(k, cc):
                pltpu.async_copy(v0_v.at[pl.ds(k * 128, 128)],
                                 acc.at[i2_v.at[k]], csem, add=True)
                pltpu.async_copy(v1_v.at[pl.ds(k * 128, 128)],
                                 acc.at[i2p1_v.at[k]], csem, add=True)
                return cc
            lax.fori_loop(0, KROWS, sb, 0)

        def drain_scatter():
            cp = pltpu.make_async_copy(h00_hbm.at[pl.ds(0, S)], v0_a, csem)
            cp.wait()
            cp.wait()

        def zbody(k, c):
            buf_v[pl.ds(k * 16, 16)] = jnp.zeros((16,), jnp.float32)
            return c
        lax.fori_loop(0, SLICE // 16, zbody, 0)
        pltpu.sync_copy(buf_v, acc.at[pl.ds(sid * SLICE, SLICE)])
        pltpu.sync_copy(v_hbm.at[pl.ds(0, NP)], v_v)
        plsc.subcore_barrier()

        fire_stage(0, bufs_a)

        def body(p, c):
            sub0 = p * 2
            drain_stage()

            @pl.when(p > 0)
            def _():
                drain_scatter()

            fire_stage(sub0 + 1, bufs_b)
            compute(bufs_a)
            fire_scatter(bufs_a)
            drain_stage()
            drain_scatter()

            @pl.when(p + 1 < half)
            def _():
                fire_stage(sub0 + 2, bufs_a)
            compute(bufs_b)
            fire_scatter(bufs_b)
            return c
        lax.fori_loop(0, half, body, 0)
        drain_scatter()

        plsc.subcore_barrier()
        pltpu.sync_copy(acc.at[pl.ds(sid * SLICE, SLICE)], buf_v)
        pltpu.sync_copy(buf_v, out_hbm.at[cid, pl.ds(sid * SLICE, SLICE)])

    return sc_matvec


def _tc_init_body(apk2_ref, b_ref, r_ref, p_ref):
    r = apk2_ref[0] + apk2_ref[1] - b_ref[...]
    r_ref[...] = r
    p_ref[...] = -r


def _tc_init(apk2, b):
    shp = jax.ShapeDtypeStruct((NP2 // 128, 128), jnp.float32)
    return pl.pallas_call(_tc_init_body, out_shape=(shp, shp))(apk2, b)


def _tc_update_body(apk2_ref, pk_ref, rk_ref, xk_ref, xo_ref, ro_ref, po_ref):
    apk = apk2_ref[0] + apk2_ref[1]
    rk = rk_ref[...]
    pk = pk_ref[...]
    rkrk = jnp.sum(rk * rk)
    alpha = rkrk / jnp.sum(pk * apk)
    xo_ref[...] = xk_ref[...] + alpha * pk
    rn = rk + alpha * apk
    ro_ref[...] = rn
    beta = jnp.sum(rn * rn) / rkrk
    po_ref[...] = -rn + beta * pk


def _tc_update(apk2, pk, rk, xk):
    shp = jax.ShapeDtypeStruct((NP2 // 128, 128), jnp.float32)
    return pl.pallas_call(_tc_update_body,
                          out_shape=(shp, shp, shp))(apk2, pk, rk, xk)


def kernel(H, B, x0, i, j):
    f32 = jnp.float32
    e = i.shape[0]
    per_w = -(-e // (NW * 2 * S)) * 2 * S
    e_pad = per_w * NW
    nsub = per_w // S
    pad = e_pad - e

    h00 = jnp.concatenate([H[:, 0, 0], jnp.zeros((pad,), f32)])
    h01 = jnp.concatenate([H[:, 0, 1], jnp.zeros((pad,), f32)])
    h10 = jnp.concatenate([H[:, 1, 0], jnp.zeros((pad,), f32)])
    h11 = jnp.concatenate([H[:, 1, 1], jnp.zeros((pad,), f32)])
    i2 = jnp.concatenate([2 * i, jnp.full((pad,), NP2 - 2, jnp.int32)])
    i2p1 = i2 + 1
    j2 = jnp.concatenate([2 * j, jnp.zeros((pad,), jnp.int32)])
    i2 = i2.reshape(e_pad // 128, 128)
    i2p1 = i2p1.reshape(e_pad // 128, 128)

    bp = jnp.pad(B, (0, NP2 - NP)).reshape(NP2 // 128, 128)
    x0p = jnp.pad(x0, (0, NP2 - NP))

    sc_matvec = _make_sc_matvec(nsub)

    def matvec(v):
        out = sc_matvec(v, j2, i2, i2p1, h00, h01, h10, h11)
        return out.reshape(2, NP2 // 128, 128)

    rk, pk = _tc_init(matvec(x0p), bp)
    xk = x0p.reshape(NP2 // 128, 128)

    def body(carry, _):
        xk, rk, pk = carry
        apk2 = matvec(pk.reshape(NP2))
        xk, rk, pk = _tc_update(apk2, pk, rk, xk)
        return (xk, rk, pk), None

    (xk, rk, pk), _ = lax.scan(body, (xk, rk, pk), None, length=CG_ITERS_K)
    return xk.reshape(NP2)[:NP]
